# baseline (speedup 1.0000x reference)
import functools

import jax
import jax.numpy as jnp
from jax import lax
from jax.experimental import pallas as pl
from jax.experimental.pallas import tpu as pltpu
from jax.experimental.pallas import tpu_sc as plsc

_N = 10000
_E = 160000
_INC = 256
_OUTC = 256
_K = 27
_EPS = 1e-5

_HALF = 128
_HCOLS = _K * _HALF
_NS = 16
_EPT = _E // _NS
_CH = 16
_G = 25
_NSUP = _EPT // (_G * _CH)
_NBUF = 5
_NPAD = 10240
_RPT = _NPAD // _NS


def _mm_body(x_ref, w_ref, o_ref):
    for kk in range(o_ref.shape[0]):
        o_ref[kk, :, :] = jnp.dot(x_ref[...], w_ref[kk],
                                  preferred_element_type=jnp.float32)


def _matmul_half(xb, wb, half):
    BN, KB = 1000, 9
    return pl.pallas_call(
        _mm_body,
        grid=(_N // BN, _K // KB),
        in_specs=[pl.BlockSpec((BN, _INC), lambda i, j: (i, 0)),
                  pl.BlockSpec((KB, _INC, _HALF),
                               lambda i, j: (j, 0, half))],
        out_specs=pl.BlockSpec((KB, BN, _HALF), lambda i, j: (j, i, 0)),
        out_shape=jax.ShapeDtypeStruct((_K, _N, _HALF), jnp.float32),
    )(xb, wb)


_sc_mesh = plsc.VectorSubcoreMesh(core_axis_name="c", subcore_axis_name="s")


@functools.partial(
    pl.kernel,
    out_type=(jax.ShapeDtypeStruct((_NPAD, _HALF), jnp.float32),
              jax.ShapeDtypeStruct((_NPAD, _HALF), jnp.float32)),
    mesh=_sc_mesh,
    scratch_types=[
        pltpu.VMEM((_G, _CH), jnp.int32),
        pltpu.VMEM((_G * _CH,), jnp.int32),
        pltpu.VMEM((_G * _CH,), jnp.int32),
        pltpu.VMEM((_G * _CH,), jnp.int32),
        pltpu.VMEM((_NBUF * _CH, _HALF), jnp.float32),
        pltpu.VMEM_SHARED((_NPAD, _HALF), jnp.float32),
        [pltpu.SemaphoreType.DMA] * _NBUF,
        [pltpu.SemaphoreType.DMA] * _NBUF,
    ],
)
def _sc_scatter(src_hbm, kidx_hbm, dst4_hbm, hlo_hbm, hhi_hbm,
                outlo_hbm, outhi_hbm,
                dlist_v, src_v, kid_v, alist_v, rows_v, acc, gsems, ssems):
    c = lax.axis_index("c")
    s = lax.axis_index("s")

    def zrow(r, _):
        for j in range(_HALF // 16):
            rows_v[r, pl.ds(16 * j, 16)] = jnp.zeros((16,), jnp.float32)
        return 0

    lax.fori_loop(0, _NBUF * _CH, zrow, 0)
    for m in range(_RPT // (_NBUF * _CH)):
        pltpu.sync_copy(
            rows_v, acc.at[pl.ds(s * _RPT + m * (_NBUF * _CH), _NBUF * _CH)])
    plsc.subcore_barrier()

    def run(table_hbm, out_hbm):
        def sup_body(si, _):
            off = s * _EPT + si * (_G * _CH)
            pltpu.sync_copy(src_hbm.at[pl.ds(off, _G * _CH)], src_v)
            pltpu.sync_copy(kidx_hbm.at[pl.ds(off, _G * _CH)], kid_v)
            pltpu.sync_copy(dst4_hbm.at[s, si], dlist_v)

            def addr_body(ci, _):
                sl = pl.ds(ci * _CH, _CH)
                alist_v[sl] = kid_v[sl] * _N + src_v[sl]
                return 0

            lax.fori_loop(0, _G, addr_body, 0)

            def grp_body(k, _):
                base = k * _NBUF
                gs = []
                for b in range(_NBUF):
                    rv = rows_v.at[pl.ds(b * _CH, _CH)]

                    @pl.when(k > 0)
                    def _(b=b, rv=rv):
                        pltpu.make_async_copy(
                            rv, acc.at[dlist_v.at[base + b]],
                            ssems[b]).wait()
                    gs.append(pltpu.async_copy(
                        table_hbm.at[alist_v.at[pl.ds((base + b) * _CH, _CH)]],
                        rv, gsems[b]))
                for b in range(_NBUF):
                    gs[b].wait()
                    pltpu.async_copy(rows_v.at[pl.ds(b * _CH, _CH)],
                                     acc.at[dlist_v.at[base + b]],
                                     ssems[b], add=True)
                return 0

            lax.fori_loop(0, _G // _NBUF, grp_body, 0)
            last = (_G // _NBUF - 1) * _NBUF
            for b in range(_NBUF):
                pltpu.make_async_copy(rows_v.at[pl.ds(b * _CH, _CH)],
                                      acc.at[dlist_v.at[last + b]],
                                      ssems[b]).wait()
            return 0

        lax.fori_loop(0, _NSUP, sup_body, 0)
        plsc.subcore_barrier()
        pltpu.sync_copy(acc.at[pl.ds(s * _RPT, _RPT)],
                        out_hbm.at[pl.ds(s * _RPT, _RPT)])

    @pl.when(c == 0)
    def _():
        run(hlo_hbm, outlo_hbm)

    @pl.when(c == 1)
    def _():
        run(hhi_hbm, outhi_hbm)


def _stats_body(lo_ref, hi_ref, sum_ref, sq_ref, acc_s, acc_q):
    i = pl.program_id(0)

    @pl.when(i == 0)
    def _():
        acc_s[...] = jnp.zeros_like(acc_s)
        acc_q[...] = jnp.zeros_like(acc_q)

    v = jnp.concatenate([lo_ref[...], hi_ref[...]], axis=1)
    acc_s[...] += jnp.sum(v, axis=0, keepdims=True)
    acc_q[...] += jnp.sum(v * v, axis=0, keepdims=True)

    @pl.when(i == pl.num_programs(0) - 1)
    def _():
        sum_ref[...] = acc_s[...]
        sq_ref[...] = acc_q[...]


def _apply_body(lo_ref, hi_ref, sum_ref, sq_ref, g_ref, b_ref, o_ref):
    mu = sum_ref[...] / _N
    var = sq_ref[...] / _N - mu * mu
    scale = g_ref[...] * lax.rsqrt(var + _EPS)
    shift = b_ref[...] - mu * scale
    ylo = lo_ref[...] * scale[:, :_HALF] + shift[:, :_HALF]
    yhi = hi_ref[...] * scale[:, _HALF:] + shift[:, _HALF:]
    o_ref[:, :_HALF] = jnp.maximum(ylo, 0.0)
    o_ref[:, _HALF:] = jnp.maximum(yhi, 0.0)


def _batchnorm_relu(out_lo, out_hi, gamma, beta):
    BS = 2000
    g2 = gamma.reshape(1, _OUTC)
    b2 = beta.reshape(1, _OUTC)
    sums, sqs = pl.pallas_call(
        _stats_body,
        grid=(_N // BS,),
        in_specs=[pl.BlockSpec((BS, _HALF), lambda i: (i, 0)),
                  pl.BlockSpec((BS, _HALF), lambda i: (i, 0))],
        out_specs=[pl.BlockSpec((1, _OUTC), lambda i: (0, 0)),
                   pl.BlockSpec((1, _OUTC), lambda i: (0, 0))],
        out_shape=[jax.ShapeDtypeStruct((1, _OUTC), jnp.float32),
                   jax.ShapeDtypeStruct((1, _OUTC), jnp.float32)],
        scratch_shapes=[pltpu.VMEM((1, _OUTC), jnp.float32),
                        pltpu.VMEM((1, _OUTC), jnp.float32)],
    )(out_lo, out_hi)
    return pl.pallas_call(
        _apply_body,
        grid=(_N // BS,),
        in_specs=[pl.BlockSpec((BS, _HALF), lambda i: (i, 0)),
                  pl.BlockSpec((BS, _HALF), lambda i: (i, 0)),
                  pl.BlockSpec((1, _OUTC), lambda i: (0, 0)),
                  pl.BlockSpec((1, _OUTC), lambda i: (0, 0)),
                  pl.BlockSpec((1, _OUTC), lambda i: (0, 0)),
                  pl.BlockSpec((1, _OUTC), lambda i: (0, 0))],
        out_specs=pl.BlockSpec((BS, _OUTC), lambda i: (i, 0)),
        out_shape=jax.ShapeDtypeStruct((_N, _OUTC), jnp.float32),
    )(out_lo, out_hi, sums, sqs, g2, b2)


def kernel(x, W, gamma, beta, edge_index, kernel_idx):
    wb = W.astype(jnp.bfloat16)
    xb = x.astype(jnp.bfloat16)
    h_lo = _matmul_half(xb, wb, 0).reshape(_K * _N, _HALF)
    h_hi = _matmul_half(xb, wb, 1).reshape(_K * _N, _HALF)
    dst4 = edge_index[1].reshape(_NS, _NSUP, _G, _CH)
    out_lo, out_hi = _sc_scatter(edge_index[0], kernel_idx, dst4,
                                 h_lo, h_hi)
    return _batchnorm_relu(out_lo, out_hi, gamma, beta)


# speedup vs baseline: 1.0124x; 1.0124x over previous
import functools

import jax
import jax.numpy as jnp
from jax import lax
from jax.experimental import pallas as pl
from jax.experimental.pallas import tpu as pltpu
from jax.experimental.pallas import tpu_sc as plsc

_N = 10000
_E = 160000
_INC = 256
_OUTC = 256
_K = 27
_EPS = 1e-5

_HALF = 128
_HCOLS = _K * _HALF
_NS = 16
_EPT = _E // _NS
_CH = 16
_G = 25
_NSUP = _EPT // (_G * _CH)
_NBUF = 5
_NPAD = 10240
_RPT = _NPAD // _NS


def _mm_body(x_ref, w_ref, o_ref):
    for kk in range(o_ref.shape[0]):
        o_ref[kk, :, :] = jnp.dot(x_ref[...], w_ref[kk],
                                  preferred_element_type=jnp.float32)


def _matmul_half(xb, wb, half):
    BN, KB = 1000, 9
    return pl.pallas_call(
        _mm_body,
        grid=(_N // BN, _K // KB),
        in_specs=[pl.BlockSpec((BN, _INC), lambda i, j: (i, 0)),
                  pl.BlockSpec((KB, _INC, _HALF),
                               lambda i, j: (j, 0, half))],
        out_specs=pl.BlockSpec((KB, BN, _HALF), lambda i, j: (j, i, 0)),
        out_shape=jax.ShapeDtypeStruct((_K, _N, _HALF), jnp.float32),
    )(xb, wb)


_sc_mesh = plsc.VectorSubcoreMesh(core_axis_name="c", subcore_axis_name="s")


@functools.partial(
    pl.kernel,
    out_type=(jax.ShapeDtypeStruct((_NPAD, _HALF), jnp.float32),
              jax.ShapeDtypeStruct((_NPAD, _HALF), jnp.float32)),
    mesh=_sc_mesh,
    scratch_types=[
        pltpu.VMEM((2, _G, _CH), jnp.int32),
        pltpu.VMEM((_NBUF * _CH, _HALF), jnp.float32),
        pltpu.VMEM_SHARED((_NPAD, _HALF), jnp.float32),
        [pltpu.SemaphoreType.DMA] * _NBUF,
        [pltpu.SemaphoreType.DMA] * _NBUF,
    ],
)
def _sc_scatter(addr_hbm, dst_hbm, hlo_hbm, hhi_hbm,
                outlo_hbm, outhi_hbm,
                ipack_v, rows_v, acc, gsems, ssems):
    c = lax.axis_index("c")
    s = lax.axis_index("s")

    def zrow(r, _):
        for j in range(_HALF // 16):
            rows_v[r, pl.ds(16 * j, 16)] = jnp.zeros((16,), jnp.float32)
        return 0

    lax.fori_loop(0, _NBUF * _CH, zrow, 0)
    for m in range(_RPT // (_NBUF * _CH)):
        pltpu.sync_copy(
            rows_v, acc.at[pl.ds(s * _RPT + m * (_NBUF * _CH), _NBUF * _CH)])
    plsc.subcore_barrier()

    def run(table_hbm, out_hbm):
        def sup_body(si, _):
            pltpu.sync_copy(addr_hbm.at[s, si], ipack_v.at[0])
            pltpu.sync_copy(dst_hbm.at[s, si], ipack_v.at[1])

            def grp_body(k, _):
                base = k * _NBUF
                gs = []
                for b in range(_NBUF):
                    rv = rows_v.at[pl.ds(b * _CH, _CH)]

                    @pl.when(k > 0)
                    def _(b=b, rv=rv):
                        pltpu.make_async_copy(
                            rv, acc.at[ipack_v.at[1, base + b]],
                            ssems[b]).wait()
                    gs.append(pltpu.async_copy(
                        table_hbm.at[ipack_v.at[0, base + b]],
                        rv, gsems[b]))
                for b in range(_NBUF):
                    gs[b].wait()
                    pltpu.async_copy(rows_v.at[pl.ds(b * _CH, _CH)],
                                     acc.at[ipack_v.at[1, base + b]],
                                     ssems[b], add=True)
                return 0

            lax.fori_loop(0, _G // _NBUF, grp_body, 0)
            last = (_G // _NBUF - 1) * _NBUF
            for b in range(_NBUF):
                pltpu.make_async_copy(rows_v.at[pl.ds(b * _CH, _CH)],
                                      acc.at[ipack_v.at[1, last + b]],
                                      ssems[b]).wait()
            return 0

        lax.fori_loop(0, _NSUP, sup_body, 0)
        plsc.subcore_barrier()
        pltpu.sync_copy(acc.at[pl.ds(s * _RPT, _RPT)],
                        out_hbm.at[pl.ds(s * _RPT, _RPT)])

    @pl.when(c == 0)
    def _():
        run(hlo_hbm, outlo_hbm)

    @pl.when(c == 1)
    def _():
        run(hhi_hbm, outhi_hbm)


def _stats_body(lo_ref, hi_ref, sum_ref, sq_ref, acc_s, acc_q):
    i = pl.program_id(0)

    @pl.when(i == 0)
    def _():
        acc_s[...] = jnp.zeros_like(acc_s)
        acc_q[...] = jnp.zeros_like(acc_q)

    v = jnp.concatenate([lo_ref[...], hi_ref[...]], axis=1)
    acc_s[...] += jnp.sum(v, axis=0, keepdims=True)
    acc_q[...] += jnp.sum(v * v, axis=0, keepdims=True)

    @pl.when(i == pl.num_programs(0) - 1)
    def _():
        sum_ref[...] = acc_s[...]
        sq_ref[...] = acc_q[...]


def _apply_body(lo_ref, hi_ref, sum_ref, sq_ref, g_ref, b_ref, o_ref):
    mu = sum_ref[...] / _N
    var = sq_ref[...] / _N - mu * mu
    scale = g_ref[...] * lax.rsqrt(var + _EPS)
    shift = b_ref[...] - mu * scale
    ylo = lo_ref[...] * scale[:, :_HALF] + shift[:, :_HALF]
    yhi = hi_ref[...] * scale[:, _HALF:] + shift[:, _HALF:]
    o_ref[:, :_HALF] = jnp.maximum(ylo, 0.0)
    o_ref[:, _HALF:] = jnp.maximum(yhi, 0.0)


def _batchnorm_relu(out_lo, out_hi, gamma, beta):
    BS = 2000
    g2 = gamma.reshape(1, _OUTC)
    b2 = beta.reshape(1, _OUTC)
    sums, sqs = pl.pallas_call(
        _stats_body,
        grid=(_N // BS,),
        in_specs=[pl.BlockSpec((BS, _HALF), lambda i: (i, 0)),
                  pl.BlockSpec((BS, _HALF), lambda i: (i, 0))],
        out_specs=[pl.BlockSpec((1, _OUTC), lambda i: (0, 0)),
                   pl.BlockSpec((1, _OUTC), lambda i: (0, 0))],
        out_shape=[jax.ShapeDtypeStruct((1, _OUTC), jnp.float32),
                   jax.ShapeDtypeStruct((1, _OUTC), jnp.float32)],
        scratch_shapes=[pltpu.VMEM((1, _OUTC), jnp.float32),
                        pltpu.VMEM((1, _OUTC), jnp.float32)],
    )(out_lo, out_hi)
    return pl.pallas_call(
        _apply_body,
        grid=(_N // BS,),
        in_specs=[pl.BlockSpec((BS, _HALF), lambda i: (i, 0)),
                  pl.BlockSpec((BS, _HALF), lambda i: (i, 0)),
                  pl.BlockSpec((1, _OUTC), lambda i: (0, 0)),
                  pl.BlockSpec((1, _OUTC), lambda i: (0, 0)),
                  pl.BlockSpec((1, _OUTC), lambda i: (0, 0)),
                  pl.BlockSpec((1, _OUTC), lambda i: (0, 0))],
        out_specs=pl.BlockSpec((BS, _OUTC), lambda i: (i, 0)),
        out_shape=jax.ShapeDtypeStruct((_N, _OUTC), jnp.float32),
    )(out_lo, out_hi, sums, sqs, g2, b2)


def kernel(x, W, gamma, beta, edge_index, kernel_idx):
    addr = kernel_idx * _N + edge_index[0]
    addr4 = addr.reshape(_NS, _NSUP, _G, _CH)
    dst4 = edge_index[1].reshape(_NS, _NSUP, _G, _CH)
    wb = W.astype(jnp.bfloat16)
    xb = x.astype(jnp.bfloat16)
    h_lo = _matmul_half(xb, wb, 0).reshape(_K * _N, _HALF)
    h_hi = _matmul_half(xb, wb, 1).reshape(_K * _N, _HALF)
    out_lo, out_hi = _sc_scatter(addr4, dst4, h_lo, h_hi)
    return _batchnorm_relu(out_lo, out_hi, gamma, beta)
